# baseline (speedup 1.0000x reference)
import jax
import jax.numpy as jnp
from jax.experimental import pallas as pl
from jax.experimental.pallas import tpu as pltpu

B = 4
L = 2048
E = 64
H = 512
VOCAB = 512
S = 4
HW = 2100
HWP = 2176
RHP = 512
N_RES = 4
EPS = 1e-05
Lp = L // S
KW = 8
T = 512
NT = L // T
TB = 16
NTB = Lp // TB
G3 = 3 * H

_f32 = jnp.float32


PR = T // S + 2


def _conv_kernel(xw_ref, emb_ref, w_ref, b_ref, out_ref):
    idx = xw_ref[0]
    oh = (idx == jax.lax.broadcasted_iota(jnp.int32, (S * PR, VOCAB), 1))
    xe = jnp.dot(oh.astype(_f32), emb_ref[:],
                 preferred_element_type=_f32)
    m = None
    for p in range(S):
        parts = []
        for d in range(KW):
            q = (p - d) % S
            s = (p - d - q) // S
            parts.append(xe[q * PR + 2 + s: q * PR + 2 + s + T // S])
        xwin = jnp.concatenate(parts, axis=1)
        y = jnp.maximum(
            jnp.dot(xwin, w_ref[:], preferred_element_type=_f32) + b_ref[:],
            0.0)
        m = y if m is None else jnp.maximum(m, y)
    out_ref[0] = m


def _res_kernel(y_ref, w1_ref, b1_ref, g_ref, bt_ref, w2_ref, b2_ref, out_ref):
    y = y_ref[:]
    for i in range(N_RES):
        r = jnp.maximum(y, 0.0)
        r = jnp.dot(r, w1_ref[i], preferred_element_type=_f32) + b1_ref[i]
        r = jnp.maximum(r, 0.0)
        r = r * g_ref[i] + bt_ref[i]
        y = y + jnp.dot(r, w2_ref[i], preferred_element_type=_f32) + b2_ref[i]
    out_ref[:] = y


def _proj_kernel(y_ref, w_ref, b_ref, out_ref):
    out_ref[:] = (jnp.dot(y_ref[:], w_ref[:], preferred_element_type=_f32)
                  + b_ref[:])


def _gru_kernel(hs0_ref, gif_ref, gib_ref, whh_ref, bhh_ref,
                outf_ref, outb_ref, hs):
    @pl.when(pl.program_id(0) == 0)
    def _():
        hs[:] = hs0_ref[:]

    h = hs[:]
    hf = h[0:B]
    hb = h[B:2 * B]
    whh = whh_ref[:]
    for i in range(TB):
        hcat = jnp.concatenate([hf, hb], axis=0)
        g = jnp.dot(hcat.astype(jnp.bfloat16), whh,
                    preferred_element_type=_f32) + bhh_ref[:]
        ghf = g[0:B, 0:G3]
        ghb = g[B:2 * B, G3:2 * G3]
        gif = gif_ref[:, i, :]
        gib = gib_ref[:, TB - 1 - i, :]

        def gates(gi, gh, hprev):
            rg = jax.nn.sigmoid(gi[:, 0:H] + gh[:, 0:H])
            zg = jax.nn.sigmoid(gi[:, H:2 * H] + gh[:, H:2 * H])
            ng = jnp.tanh(gi[:, 2 * H:3 * H] + rg * gh[:, 2 * H:3 * H])
            return (1.0 - zg) * ng + zg * hprev

        hf = gates(gif, ghf, hf)
        hb = gates(gib, ghb, hb)
        outf_ref[:, i, :] = hf
        outb_ref[:, TB - 1 - i, :] = hb
    hs[:] = jnp.concatenate([hf, hb], axis=0)


def kernel(x, h, emb, conv_params, res_params, gru_params):
    Wc = jnp.zeros((KW * E, HWP), _f32)
    bc = jnp.zeros((1, HWP), _f32)
    off = 0
    for i, (W, b) in enumerate(conv_params):
        nf = W.shape[0]
        for d in range(i + 1):
            Wc = Wc.at[d * E:(d + 1) * E, off:off + nf].set(W[:, 0, i - d, :].T)
        bc = bc.at[0, off:off + nf].set(b)
        off += nf

    import numpy as _np
    _jj = _np.arange(PR)
    _pos = (_np.arange(NT)[:, None, None] * T
            + 4 * (_jj[None, None, :] - 2) + _np.arange(S)[None, :, None])
    xp = jnp.pad(x.astype(jnp.int32), ((0, 0), (KW, 0)), constant_values=VOCAB)
    xw = jnp.take(xp, jnp.asarray(_pos.reshape(-1) + KW), axis=1)
    xw = xw.reshape(B * NT, S * PR, 1)

    Yp = pl.pallas_call(
        _conv_kernel,
        grid=(B * NT,),
        in_specs=[
            pl.BlockSpec((1, S * PR, 1), lambda g: (g, 0, 0)),
            pl.BlockSpec((VOCAB, E), lambda g: (0, 0)),
            pl.BlockSpec((KW * E, HWP), lambda g: (0, 0)),
            pl.BlockSpec((1, HWP), lambda g: (0, 0)),
        ],
        out_specs=pl.BlockSpec((1, T // S, HWP), lambda g: (g, 0, 0)),
        out_shape=jax.ShapeDtypeStruct((B * NT, T // S, HWP), _f32),
    )(xw, emb, Wc, bc)
    Yf = Yp.reshape(B * Lp, HWP)

    gm = 1.0 / jnp.sqrt(1.0 + EPS)
    w1 = jnp.stack([jnp.zeros((HWP, RHP), _f32).at[:HW, :400].set(p[0].T)
                    for p in res_params])
    b1 = jnp.stack([jnp.zeros((1, RHP), _f32).at[0, :400].set(p[1])
                    for p in res_params])
    gmul = jnp.stack([jnp.zeros((1, RHP), _f32).at[0, :400].set(p[4] * gm)
                      for p in res_params])
    beta = jnp.stack([jnp.zeros((1, RHP), _f32).at[0, :400].set(p[5])
                      for p in res_params])
    w2 = jnp.stack([jnp.zeros((RHP, HWP), _f32).at[:400, :HW].set(p[2].T)
                    for p in res_params])
    b2 = jnp.stack([jnp.zeros((1, HWP), _f32).at[0, :HW].set(p[3])
                    for p in res_params])

    RM = 256
    Yr = pl.pallas_call(
        _res_kernel,
        grid=(B * Lp // RM,),
        in_specs=[
            pl.BlockSpec((RM, HWP), lambda m: (m, 0)),
            pl.BlockSpec((N_RES, HWP, RHP), lambda m: (0, 0, 0)),
            pl.BlockSpec((N_RES, 1, RHP), lambda m: (0, 0, 0)),
            pl.BlockSpec((N_RES, 1, RHP), lambda m: (0, 0, 0)),
            pl.BlockSpec((N_RES, 1, RHP), lambda m: (0, 0, 0)),
            pl.BlockSpec((N_RES, RHP, HWP), lambda m: (0, 0, 0)),
            pl.BlockSpec((N_RES, 1, HWP), lambda m: (0, 0, 0)),
        ],
        out_specs=pl.BlockSpec((RM, HWP), lambda m: (m, 0)),
        out_shape=jax.ShapeDtypeStruct((B * Lp, HWP), _f32),
    )(Yf, w1, b1, gmul, beta, w2, b2)

    Wih_f, Whh_f, bih_f, bhh_f = gru_params[0]
    Wih_b, Whh_b, bih_b, bhh_b = gru_params[1]
    Wih = jnp.concatenate(
        [jnp.zeros((HWP, G3), _f32).at[:HW, :].set(Wih_f.T),
         jnp.zeros((HWP, G3), _f32).at[:HW, :].set(Wih_b.T)], axis=1)
    bih = jnp.concatenate([bih_f, bih_b])[None, :]

    Gi = pl.pallas_call(
        _proj_kernel,
        grid=(B * Lp // RM,),
        in_specs=[
            pl.BlockSpec((RM, HWP), lambda m: (m, 0)),
            pl.BlockSpec((HWP, 2 * G3), lambda m: (0, 0)),
            pl.BlockSpec((1, 2 * G3), lambda m: (0, 0)),
        ],
        out_specs=pl.BlockSpec((RM, 2 * G3), lambda m: (m, 0)),
        out_shape=jax.ShapeDtypeStruct((B * Lp, 2 * G3), _f32),
    )(Yr, Wih, bih)
    Gi = Gi.reshape(B, Lp, 2 * G3)

    Whh = jnp.concatenate([Whh_f.T, Whh_b.T],
                          axis=1).astype(jnp.bfloat16)
    bhh = jnp.concatenate([bhh_f, bhh_b])[None, :]
    hs0 = jnp.concatenate([h[0], h[1]], axis=0)

    ysf, ysb = pl.pallas_call(
        _gru_kernel,
        grid=(NTB,),
        in_specs=[
            pl.BlockSpec((2 * B, H), lambda t: (0, 0)),
            pl.BlockSpec((B, TB, G3), lambda t: (0, t, 0)),
            pl.BlockSpec((B, TB, G3), lambda t: (0, NTB - 1 - t, 1)),
            pl.BlockSpec((H, 2 * G3), lambda t: (0, 0)),
            pl.BlockSpec((1, 2 * G3), lambda t: (0, 0)),
        ],
        out_specs=[
            pl.BlockSpec((B, TB, H), lambda t: (0, t, 0)),
            pl.BlockSpec((B, TB, H), lambda t: (0, NTB - 1 - t, 0)),
        ],
        out_shape=[
            jax.ShapeDtypeStruct((B, Lp, H), _f32),
            jax.ShapeDtypeStruct((B, Lp, H), _f32),
        ],
        scratch_shapes=[pltpu.VMEM((2 * B, H), _f32)],
    )(hs0, Gi, Gi, Whh, bhh)

    out = jnp.concatenate([ysf, ysb], axis=-1)
    hn = jnp.stack([ysf[:, -1, :], ysb[:, 0, :]], axis=0)
    return out, hn


# speedup vs baseline: 1.0104x; 1.0104x over previous
import jax
import jax.numpy as jnp
from jax.experimental import pallas as pl
from jax.experimental.pallas import tpu as pltpu

B = 4
L = 2048
E = 64
H = 512
VOCAB = 512
S = 4
HW = 2100
HWP = 2176
RHP = 512
N_RES = 4
EPS = 1e-05
Lp = L // S
KW = 8
T = 512
NT = L // T
TB = 16
NTB = Lp // TB
G3 = 3 * H

_f32 = jnp.float32


PR = T // S + 2


def _conv_kernel(xw_ref, emb_ref, w_ref, b_ref, out_ref):
    idx = xw_ref[0]
    oh = (idx == jax.lax.broadcasted_iota(jnp.int32, (S * PR, VOCAB), 1))
    xe = jnp.dot(oh.astype(_f32), emb_ref[:],
                 preferred_element_type=_f32)
    m = None
    for p in range(S):
        parts = []
        for d in range(KW):
            q = (p - d) % S
            s = (p - d - q) // S
            parts.append(xe[q * PR + 2 + s: q * PR + 2 + s + T // S])
        xwin = jnp.concatenate(parts, axis=1)
        y = jnp.maximum(
            jnp.dot(xwin, w_ref[:], preferred_element_type=_f32) + b_ref[:],
            0.0)
        m = y if m is None else jnp.maximum(m, y)
    out_ref[0] = m


def _res_kernel(y_ref, w1_ref, b1_ref, g_ref, bt_ref, w2_ref, b2_ref, out_ref):
    y = y_ref[:]
    for i in range(N_RES):
        r = jnp.maximum(y, 0.0)
        r = jnp.dot(r, w1_ref[i], preferred_element_type=_f32) + b1_ref[i]
        r = jnp.maximum(r, 0.0)
        r = r * g_ref[i] + bt_ref[i]
        y = y + jnp.dot(r, w2_ref[i], preferred_element_type=_f32) + b2_ref[i]
    out_ref[:] = y


def _proj_kernel(y_ref, w_ref, b_ref, out_ref):
    out_ref[:] = (jnp.dot(y_ref[:], w_ref[:], preferred_element_type=_f32)
                  + b_ref[:])


def _gru_kernel(hs0_ref, gif_ref, gib_ref, whh_ref, bhh_ref,
                outf_ref, outb_ref, hs):
    zb = jnp.zeros((B, H), _f32)

    def widen(hcat):
        return jnp.concatenate(
            [jnp.concatenate([hcat[0:B], zb], axis=1),
             jnp.concatenate([zb, hcat[B:2 * B]], axis=1)], axis=0)

    @pl.when(pl.program_id(0) == 0)
    def _():
        hs[:] = widen(hs0_ref[:])

    hwide = hs[:]
    whh = whh_ref[:]
    bhh = bhh_ref[:]
    for i in range(TB):
        gh = jnp.dot(hwide.astype(jnp.bfloat16), whh,
                     preferred_element_type=_f32) + bhh
        gi = jnp.concatenate(
            [gif_ref[:, i, :], gib_ref[:, TB - 1 - i, :]], axis=0)
        rg = jax.nn.sigmoid(gi[:, 0:H] + gh[:, 0:H])
        zg = jax.nn.sigmoid(gi[:, H:2 * H] + gh[:, H:2 * H])
        ng = jnp.tanh(gi[:, 2 * H:3 * H] + rg * gh[:, 2 * H:3 * H])
        hprev = hwide[0:2 * B, 0:H] + hwide[0:2 * B, H:2 * H]
        hnew = (1.0 - zg) * ng + zg * hprev
        outf_ref[:, i, :] = hnew[0:B]
        outb_ref[:, TB - 1 - i, :] = hnew[B:2 * B]
        hwide = widen(hnew)
    hs[:] = hwide


def kernel(x, h, emb, conv_params, res_params, gru_params):
    Wc = jnp.zeros((KW * E, HWP), _f32)
    bc = jnp.zeros((1, HWP), _f32)
    off = 0
    for i, (W, b) in enumerate(conv_params):
        nf = W.shape[0]
        for d in range(i + 1):
            Wc = Wc.at[d * E:(d + 1) * E, off:off + nf].set(W[:, 0, i - d, :].T)
        bc = bc.at[0, off:off + nf].set(b)
        off += nf

    import numpy as _np
    _jj = _np.arange(PR)
    _pos = (_np.arange(NT)[:, None, None] * T
            + 4 * (_jj[None, None, :] - 2) + _np.arange(S)[None, :, None])
    xp = jnp.pad(x.astype(jnp.int32), ((0, 0), (KW, 0)), constant_values=VOCAB)
    xw = jnp.take(xp, jnp.asarray(_pos.reshape(-1) + KW), axis=1)
    xw = xw.reshape(B * NT, S * PR, 1)

    Yp = pl.pallas_call(
        _conv_kernel,
        grid=(B * NT,),
        in_specs=[
            pl.BlockSpec((1, S * PR, 1), lambda g: (g, 0, 0)),
            pl.BlockSpec((VOCAB, E), lambda g: (0, 0)),
            pl.BlockSpec((KW * E, HWP), lambda g: (0, 0)),
            pl.BlockSpec((1, HWP), lambda g: (0, 0)),
        ],
        out_specs=pl.BlockSpec((1, T // S, HWP), lambda g: (g, 0, 0)),
        out_shape=jax.ShapeDtypeStruct((B * NT, T // S, HWP), _f32),
    )(xw, emb, Wc, bc)
    Yf = Yp.reshape(B * Lp, HWP)

    gm = 1.0 / jnp.sqrt(1.0 + EPS)
    w1 = jnp.stack([jnp.zeros((HWP, RHP), _f32).at[:HW, :400].set(p[0].T)
                    for p in res_params])
    b1 = jnp.stack([jnp.zeros((1, RHP), _f32).at[0, :400].set(p[1])
                    for p in res_params])
    gmul = jnp.stack([jnp.zeros((1, RHP), _f32).at[0, :400].set(p[4] * gm)
                      for p in res_params])
    beta = jnp.stack([jnp.zeros((1, RHP), _f32).at[0, :400].set(p[5])
                      for p in res_params])
    w2 = jnp.stack([jnp.zeros((RHP, HWP), _f32).at[:400, :HW].set(p[2].T)
                    for p in res_params])
    b2 = jnp.stack([jnp.zeros((1, HWP), _f32).at[0, :HW].set(p[3])
                    for p in res_params])

    RM = 256
    Yr = pl.pallas_call(
        _res_kernel,
        grid=(B * Lp // RM,),
        in_specs=[
            pl.BlockSpec((RM, HWP), lambda m: (m, 0)),
            pl.BlockSpec((N_RES, HWP, RHP), lambda m: (0, 0, 0)),
            pl.BlockSpec((N_RES, 1, RHP), lambda m: (0, 0, 0)),
            pl.BlockSpec((N_RES, 1, RHP), lambda m: (0, 0, 0)),
            pl.BlockSpec((N_RES, 1, RHP), lambda m: (0, 0, 0)),
            pl.BlockSpec((N_RES, RHP, HWP), lambda m: (0, 0, 0)),
            pl.BlockSpec((N_RES, 1, HWP), lambda m: (0, 0, 0)),
        ],
        out_specs=pl.BlockSpec((RM, HWP), lambda m: (m, 0)),
        out_shape=jax.ShapeDtypeStruct((B * Lp, HWP), _f32),
    )(Yf, w1, b1, gmul, beta, w2, b2)

    Wih_f, Whh_f, bih_f, bhh_f = gru_params[0]
    Wih_b, Whh_b, bih_b, bhh_b = gru_params[1]
    Wih = jnp.concatenate(
        [jnp.zeros((HWP, G3), _f32).at[:HW, :].set(Wih_f.T),
         jnp.zeros((HWP, G3), _f32).at[:HW, :].set(Wih_b.T)], axis=1)
    bih = jnp.concatenate([bih_f, bih_b])[None, :]

    Gi = pl.pallas_call(
        _proj_kernel,
        grid=(B * Lp // RM,),
        in_specs=[
            pl.BlockSpec((RM, HWP), lambda m: (m, 0)),
            pl.BlockSpec((HWP, 2 * G3), lambda m: (0, 0)),
            pl.BlockSpec((1, 2 * G3), lambda m: (0, 0)),
        ],
        out_specs=pl.BlockSpec((RM, 2 * G3), lambda m: (m, 0)),
        out_shape=jax.ShapeDtypeStruct((B * Lp, 2 * G3), _f32),
    )(Yr, Wih, bih)
    Gi = Gi.reshape(B, Lp, 2 * G3)

    Whh = jnp.concatenate([Whh_f.T, Whh_b.T],
                          axis=0).astype(jnp.bfloat16)
    bhh = jnp.concatenate([jnp.tile(bhh_f[None, :], (B, 1)),
                           jnp.tile(bhh_b[None, :], (B, 1))], axis=0)
    hs0 = jnp.concatenate([h[0], h[1]], axis=0)

    ysf, ysb = pl.pallas_call(
        _gru_kernel,
        grid=(NTB,),
        in_specs=[
            pl.BlockSpec((2 * B, H), lambda t: (0, 0)),
            pl.BlockSpec((B, TB, G3), lambda t: (0, t, 0)),
            pl.BlockSpec((B, TB, G3), lambda t: (0, NTB - 1 - t, 1)),
            pl.BlockSpec((2 * H, G3), lambda t: (0, 0)),
            pl.BlockSpec((2 * B, G3), lambda t: (0, 0)),
        ],
        out_specs=[
            pl.BlockSpec((B, TB, H), lambda t: (0, t, 0)),
            pl.BlockSpec((B, TB, H), lambda t: (0, NTB - 1 - t, 0)),
        ],
        out_shape=[
            jax.ShapeDtypeStruct((B, Lp, H), _f32),
            jax.ShapeDtypeStruct((B, Lp, H), _f32),
        ],
        scratch_shapes=[pltpu.VMEM((2 * B, 2 * H), _f32)],
    )(hs0, Gi, Gi, Whh, bhh)

    out = jnp.concatenate([ysf, ysb], axis=-1)
    hn = jnp.stack([ysf[:, -1, :], ysb[:, 0, :]], axis=0)
    return out, hn


# speedup vs baseline: 1.2212x; 1.2086x over previous
import jax
import jax.numpy as jnp
from jax.experimental import pallas as pl
from jax.experimental.pallas import tpu as pltpu

B = 4
L = 2048
E = 64
H = 512
VOCAB = 512
S = 4
HW = 2100
HWP = 2176
RHP = 512
N_RES = 4
EPS = 1e-05
Lp = L // S
KW = 8
T = 512
NT = L // T
TB = 16
NTB = Lp // TB
G3 = 3 * H

_f32 = jnp.float32


PR = T // S + 2


def _conv_kernel(xw_ref, emb_ref, w_ref, b_ref, out_ref):
    idx = xw_ref[0]
    oh = (idx == jax.lax.broadcasted_iota(jnp.int32, (S * PR, VOCAB), 1))
    xe = jnp.dot(oh.astype(_f32), emb_ref[:],
                 preferred_element_type=_f32)
    m = None
    for p in range(S):
        parts = []
        for d in range(KW):
            q = (p - d) % S
            s = (p - d - q) // S
            parts.append(xe[q * PR + 2 + s: q * PR + 2 + s + T // S])
        xwin = jnp.concatenate(parts, axis=1)
        y = jnp.maximum(
            jnp.dot(xwin, w_ref[:], preferred_element_type=_f32) + b_ref[:],
            0.0)
        m = y if m is None else jnp.maximum(m, y)
    out_ref[0] = m


def _res_kernel(y_ref, w1_ref, b1_ref, g_ref, bt_ref, w2_ref, b2_ref, out_ref):
    y = y_ref[:]
    for i in range(N_RES):
        r = jnp.maximum(y, 0.0)
        r = jnp.dot(r, w1_ref[i], preferred_element_type=_f32) + b1_ref[i]
        r = jnp.maximum(r, 0.0)
        r = r * g_ref[i] + bt_ref[i]
        y = y + jnp.dot(r, w2_ref[i], preferred_element_type=_f32) + b2_ref[i]
    out_ref[:] = y


def _proj_kernel(y_ref, w_ref, b_ref, out_ref):
    out_ref[:] = (jnp.dot(y_ref[:], w_ref[:], preferred_element_type=_f32)
                  + b_ref[:])


def _gru_kernel(hs0_ref, gif_ref, gib_ref, whf_ref, whb_ref, bn_ref,
                outf_ref, outb_ref, hs):
    @pl.when(pl.program_id(0) == 0)
    def _():
        hs[:] = hs0_ref[:]

    h = hs[:]
    hf = h[0:B]
    hb = h[B:2 * B]
    whf = whf_ref[:]
    whb = whb_ref[:]
    bnf = bn_ref[0]
    bnb = bn_ref[1]

    def gates(gi, mm, bn, hprev):
        rg = jax.nn.sigmoid(gi[:, 0:H] + mm[:, 0:H])
        zg = jax.nn.sigmoid(gi[:, H:2 * H] + mm[:, H:2 * H])
        ng = jnp.tanh(gi[:, 2 * H:3 * H] + rg * (mm[:, 2 * H:3 * H] + bn))
        return (1.0 - zg) * ng + zg * hprev

    for i in range(TB):
        mmf = jnp.dot(hf.astype(jnp.bfloat16), whf, preferred_element_type=_f32)
        mmb = jnp.dot(hb.astype(jnp.bfloat16), whb, preferred_element_type=_f32)
        hf = gates(gif_ref[:, i, :], mmf, bnf, hf)
        hb = gates(gib_ref[:, TB - 1 - i, :], mmb, bnb, hb)
        outf_ref[:, i, :] = hf
        outb_ref[:, TB - 1 - i, :] = hb
    hs[:] = jnp.concatenate([hf, hb], axis=0)


def kernel(x, h, emb, conv_params, res_params, gru_params):
    Wc = jnp.zeros((KW * E, HWP), _f32)
    bc = jnp.zeros((1, HWP), _f32)
    off = 0
    for i, (W, b) in enumerate(conv_params):
        nf = W.shape[0]
        for d in range(i + 1):
            Wc = Wc.at[d * E:(d + 1) * E, off:off + nf].set(W[:, 0, i - d, :].T)
        bc = bc.at[0, off:off + nf].set(b)
        off += nf

    xp = jnp.pad(x.astype(jnp.int32), ((0, 0), (KW, 0)), constant_values=VOCAB)
    arr = xp.reshape(B, (L + KW) // S, S)
    xw = jnp.stack([arr[:, (T // S) * n:(T // S) * n + PR] for n in range(NT)],
                   axis=1)
    xw = xw.transpose(0, 1, 3, 2).reshape(B * NT, S * PR, 1)

    Yp = pl.pallas_call(
        _conv_kernel,
        grid=(B * NT,),
        in_specs=[
            pl.BlockSpec((1, S * PR, 1), lambda g: (g, 0, 0)),
            pl.BlockSpec((VOCAB, E), lambda g: (0, 0)),
            pl.BlockSpec((KW * E, HWP), lambda g: (0, 0)),
            pl.BlockSpec((1, HWP), lambda g: (0, 0)),
        ],
        out_specs=pl.BlockSpec((1, T // S, HWP), lambda g: (g, 0, 0)),
        out_shape=jax.ShapeDtypeStruct((B * NT, T // S, HWP), _f32),
    )(xw, emb, Wc, bc)
    Yf = Yp.reshape(B * Lp, HWP)

    gm = 1.0 / jnp.sqrt(1.0 + EPS)
    w1 = jnp.stack([jnp.zeros((HWP, RHP), _f32).at[:HW, :400].set(p[0].T)
                    for p in res_params])
    b1 = jnp.stack([jnp.zeros((1, RHP), _f32).at[0, :400].set(p[1])
                    for p in res_params])
    gmul = jnp.stack([jnp.zeros((1, RHP), _f32).at[0, :400].set(p[4] * gm)
                      for p in res_params])
    beta = jnp.stack([jnp.zeros((1, RHP), _f32).at[0, :400].set(p[5])
                      for p in res_params])
    w2 = jnp.stack([jnp.zeros((RHP, HWP), _f32).at[:400, :HW].set(p[2].T)
                    for p in res_params])
    b2 = jnp.stack([jnp.zeros((1, HWP), _f32).at[0, :HW].set(p[3])
                    for p in res_params])

    RM = 256
    Yr = pl.pallas_call(
        _res_kernel,
        grid=(B * Lp // RM,),
        in_specs=[
            pl.BlockSpec((RM, HWP), lambda m: (m, 0)),
            pl.BlockSpec((N_RES, HWP, RHP), lambda m: (0, 0, 0)),
            pl.BlockSpec((N_RES, 1, RHP), lambda m: (0, 0, 0)),
            pl.BlockSpec((N_RES, 1, RHP), lambda m: (0, 0, 0)),
            pl.BlockSpec((N_RES, 1, RHP), lambda m: (0, 0, 0)),
            pl.BlockSpec((N_RES, RHP, HWP), lambda m: (0, 0, 0)),
            pl.BlockSpec((N_RES, 1, HWP), lambda m: (0, 0, 0)),
        ],
        out_specs=pl.BlockSpec((RM, HWP), lambda m: (m, 0)),
        out_shape=jax.ShapeDtypeStruct((B * Lp, HWP), _f32),
    )(Yf, w1, b1, gmul, beta, w2, b2)

    Wih_f, Whh_f, bih_f, bhh_f = gru_params[0]
    Wih_b, Whh_b, bih_b, bhh_b = gru_params[1]
    Wih = jnp.concatenate(
        [jnp.zeros((HWP, G3), _f32).at[:HW, :].set(Wih_f.T),
         jnp.zeros((HWP, G3), _f32).at[:HW, :].set(Wih_b.T)], axis=1)
    zH = jnp.zeros((H,), _f32)
    bih = jnp.concatenate(
        [bih_f + jnp.concatenate([bhh_f[0:2 * H], zH]),
         bih_b + jnp.concatenate([bhh_b[0:2 * H], zH])])[None, :]

    Gi = pl.pallas_call(
        _proj_kernel,
        grid=(B * Lp // RM,),
        in_specs=[
            pl.BlockSpec((RM, HWP), lambda m: (m, 0)),
            pl.BlockSpec((HWP, 2 * G3), lambda m: (0, 0)),
            pl.BlockSpec((1, 2 * G3), lambda m: (0, 0)),
        ],
        out_specs=pl.BlockSpec((RM, 2 * G3), lambda m: (m, 0)),
        out_shape=jax.ShapeDtypeStruct((B * Lp, 2 * G3), _f32),
    )(Yr, Wih, bih)
    Gi = Gi.reshape(B, Lp, 2 * G3)

    Whf = Whh_f.T.astype(jnp.bfloat16)
    Whb = Whh_b.T.astype(jnp.bfloat16)
    bn2 = jnp.stack([bhh_f[2 * H:3 * H][None, :],
                     bhh_b[2 * H:3 * H][None, :]])
    hs0 = jnp.concatenate([h[0], h[1]], axis=0)

    ysf, ysb = pl.pallas_call(
        _gru_kernel,
        grid=(NTB,),
        in_specs=[
            pl.BlockSpec((2 * B, H), lambda t: (0, 0)),
            pl.BlockSpec((B, TB, G3), lambda t: (0, t, 0)),
            pl.BlockSpec((B, TB, G3), lambda t: (0, NTB - 1 - t, 1)),
            pl.BlockSpec((H, G3), lambda t: (0, 0)),
            pl.BlockSpec((H, G3), lambda t: (0, 0)),
            pl.BlockSpec((2, 1, H), lambda t: (0, 0, 0)),
        ],
        out_specs=[
            pl.BlockSpec((B, TB, H), lambda t: (0, t, 0)),
            pl.BlockSpec((B, TB, H), lambda t: (0, NTB - 1 - t, 0)),
        ],
        out_shape=[
            jax.ShapeDtypeStruct((B, Lp, H), _f32),
            jax.ShapeDtypeStruct((B, Lp, H), _f32),
        ],
        scratch_shapes=[pltpu.VMEM((2 * B, H), _f32)],
    )(hs0, Gi, Gi, Whf, Whb, bn2)

    out = jnp.concatenate([ysf, ysb], axis=-1)
    hn = jnp.stack([ysf[:, -1, :], ysb[:, 0, :]], axis=0)
    return out, hn


# speedup vs baseline: 1.2512x; 1.0246x over previous
import jax
import jax.numpy as jnp
from jax.experimental import pallas as pl
from jax.experimental.pallas import tpu as pltpu

B = 4
L = 2048
E = 64
H = 512
VOCAB = 512
S = 4
HW = 2100
HWP = 2176
RHP = 512
N_RES = 4
EPS = 1e-05
Lp = L // S
KW = 8
T = 512
NT = L // T
TB = 32
NTB = Lp // TB
G3 = 3 * H

_f32 = jnp.float32


PR = T // S + 2


def _conv_kernel(xw_ref, emb_ref, w_ref, b_ref, out_ref):
    idx = xw_ref[0]
    oh = (idx == jax.lax.broadcasted_iota(jnp.int32, (S * PR, VOCAB), 1))
    xe = jnp.dot(oh.astype(_f32), emb_ref[:],
                 preferred_element_type=_f32)
    m = None
    for p in range(S):
        parts = []
        for d in range(KW):
            q = (p - d) % S
            s = (p - d - q) // S
            parts.append(xe[q * PR + 2 + s: q * PR + 2 + s + T // S])
        xwin = jnp.concatenate(parts, axis=1)
        y = jnp.maximum(
            jnp.dot(xwin, w_ref[:], preferred_element_type=_f32) + b_ref[:],
            0.0)
        m = y if m is None else jnp.maximum(m, y)
    out_ref[0] = m


def _res_kernel(y_ref, w1_ref, b1_ref, g_ref, bt_ref, w2_ref, b2_ref, out_ref):
    y = y_ref[:]
    for i in range(N_RES):
        r = jnp.maximum(y, 0.0).astype(jnp.bfloat16)
        r = jnp.dot(r, w1_ref[i], preferred_element_type=_f32) + b1_ref[i]
        r = jnp.maximum(r, 0.0)
        r = (r * g_ref[i] + bt_ref[i]).astype(jnp.bfloat16)
        y = y + jnp.dot(r, w2_ref[i], preferred_element_type=_f32) + b2_ref[i]
    out_ref[:] = y


def _proj_kernel(y_ref, w_ref, b_ref, out_ref):
    out_ref[:] = (jnp.dot(y_ref[:].astype(jnp.bfloat16), w_ref[:],
                          preferred_element_type=_f32) + b_ref[:])


def _gru_kernel(hs0_ref, gif_ref, gib_ref, whf_ref, whb_ref, bn_ref,
                outf_ref, outb_ref, hs):
    @pl.when(pl.program_id(0) == 0)
    def _():
        hs[:] = hs0_ref[:]

    h = hs[:]
    hf = h[0:B]
    hb = h[B:2 * B]
    whf = whf_ref[:]
    whb = whb_ref[:]
    bnf = bn_ref[0]
    bnb = bn_ref[1]

    def gates(gi, mm, bn, hprev):
        rg = jax.nn.sigmoid(gi[:, 0:H] + mm[:, 0:H])
        zg = jax.nn.sigmoid(gi[:, H:2 * H] + mm[:, H:2 * H])
        ng = jnp.tanh(gi[:, 2 * H:3 * H] + rg * (mm[:, 2 * H:3 * H] + bn))
        return (1.0 - zg) * ng + zg * hprev

    for i in range(TB):
        mmf = jnp.dot(hf.astype(jnp.bfloat16), whf, preferred_element_type=_f32)
        mmb = jnp.dot(hb.astype(jnp.bfloat16), whb, preferred_element_type=_f32)
        hf = gates(gif_ref[:, i, :], mmf, bnf, hf)
        hb = gates(gib_ref[:, TB - 1 - i, :], mmb, bnb, hb)
        outf_ref[:, i, :] = hf
        outb_ref[:, TB - 1 - i, :] = hb
    hs[:] = jnp.concatenate([hf, hb], axis=0)


def kernel(x, h, emb, conv_params, res_params, gru_params):
    Wc = jnp.zeros((KW * E, HWP), _f32)
    bc = jnp.zeros((1, HWP), _f32)
    off = 0
    for i, (W, b) in enumerate(conv_params):
        nf = W.shape[0]
        for d in range(i + 1):
            Wc = Wc.at[d * E:(d + 1) * E, off:off + nf].set(W[:, 0, i - d, :].T)
        bc = bc.at[0, off:off + nf].set(b)
        off += nf

    xp = jnp.pad(x.astype(jnp.int32), ((0, 0), (KW, 0)), constant_values=VOCAB)
    arr = xp.reshape(B, (L + KW) // S, S)
    xw = jnp.stack([arr[:, (T // S) * n:(T // S) * n + PR] for n in range(NT)],
                   axis=1)
    xw = xw.transpose(0, 1, 3, 2).reshape(B * NT, S * PR, 1)

    Yp = pl.pallas_call(
        _conv_kernel,
        grid=(B * NT,),
        in_specs=[
            pl.BlockSpec((1, S * PR, 1), lambda g: (g, 0, 0)),
            pl.BlockSpec((VOCAB, E), lambda g: (0, 0)),
            pl.BlockSpec((KW * E, HWP), lambda g: (0, 0)),
            pl.BlockSpec((1, HWP), lambda g: (0, 0)),
        ],
        out_specs=pl.BlockSpec((1, T // S, HWP), lambda g: (g, 0, 0)),
        out_shape=jax.ShapeDtypeStruct((B * NT, T // S, HWP), _f32),
    )(xw, emb, Wc, bc)
    Yf = Yp.reshape(B * Lp, HWP)

    gm = 1.0 / jnp.sqrt(1.0 + EPS)
    w1 = jnp.stack([jnp.zeros((HWP, RHP), _f32).at[:HW, :400].set(p[0].T)
                    for p in res_params]).astype(jnp.bfloat16)
    b1 = jnp.stack([jnp.zeros((1, RHP), _f32).at[0, :400].set(p[1])
                    for p in res_params])
    gmul = jnp.stack([jnp.zeros((1, RHP), _f32).at[0, :400].set(p[4] * gm)
                      for p in res_params])
    beta = jnp.stack([jnp.zeros((1, RHP), _f32).at[0, :400].set(p[5])
                      for p in res_params])
    w2 = jnp.stack([jnp.zeros((RHP, HWP), _f32).at[:400, :HW].set(p[2].T)
                    for p in res_params]).astype(jnp.bfloat16)
    b2 = jnp.stack([jnp.zeros((1, HWP), _f32).at[0, :HW].set(p[3])
                    for p in res_params])

    RM = 256
    Yr = pl.pallas_call(
        _res_kernel,
        grid=(B * Lp // RM,),
        in_specs=[
            pl.BlockSpec((RM, HWP), lambda m: (m, 0)),
            pl.BlockSpec((N_RES, HWP, RHP), lambda m: (0, 0, 0)),
            pl.BlockSpec((N_RES, 1, RHP), lambda m: (0, 0, 0)),
            pl.BlockSpec((N_RES, 1, RHP), lambda m: (0, 0, 0)),
            pl.BlockSpec((N_RES, 1, RHP), lambda m: (0, 0, 0)),
            pl.BlockSpec((N_RES, RHP, HWP), lambda m: (0, 0, 0)),
            pl.BlockSpec((N_RES, 1, HWP), lambda m: (0, 0, 0)),
        ],
        out_specs=pl.BlockSpec((RM, HWP), lambda m: (m, 0)),
        out_shape=jax.ShapeDtypeStruct((B * Lp, HWP), _f32),
    )(Yf, w1, b1, gmul, beta, w2, b2)

    Wih_f, Whh_f, bih_f, bhh_f = gru_params[0]
    Wih_b, Whh_b, bih_b, bhh_b = gru_params[1]
    Wih = jnp.concatenate(
        [jnp.zeros((HWP, G3), _f32).at[:HW, :].set(Wih_f.T),
         jnp.zeros((HWP, G3), _f32).at[:HW, :].set(Wih_b.T)],
        axis=1).astype(jnp.bfloat16)
    zH = jnp.zeros((H,), _f32)
    bih = jnp.concatenate(
        [bih_f + jnp.concatenate([bhh_f[0:2 * H], zH]),
         bih_b + jnp.concatenate([bhh_b[0:2 * H], zH])])[None, :]

    Gi = pl.pallas_call(
        _proj_kernel,
        grid=(B * Lp // RM,),
        in_specs=[
            pl.BlockSpec((RM, HWP), lambda m: (m, 0)),
            pl.BlockSpec((HWP, 2 * G3), lambda m: (0, 0)),
            pl.BlockSpec((1, 2 * G3), lambda m: (0, 0)),
        ],
        out_specs=pl.BlockSpec((RM, 2 * G3), lambda m: (m, 0)),
        out_shape=jax.ShapeDtypeStruct((B * Lp, 2 * G3), _f32),
    )(Yr, Wih, bih)
    Gi = Gi.reshape(B, Lp, 2 * G3)

    Whf = Whh_f.T.astype(jnp.bfloat16)
    Whb = Whh_b.T.astype(jnp.bfloat16)
    bn2 = jnp.stack([bhh_f[2 * H:3 * H][None, :],
                     bhh_b[2 * H:3 * H][None, :]])
    hs0 = jnp.concatenate([h[0], h[1]], axis=0)

    ysf, ysb = pl.pallas_call(
        _gru_kernel,
        grid=(NTB,),
        in_specs=[
            pl.BlockSpec((2 * B, H), lambda t: (0, 0)),
            pl.BlockSpec((B, TB, G3), lambda t: (0, t, 0)),
            pl.BlockSpec((B, TB, G3), lambda t: (0, NTB - 1 - t, 1)),
            pl.BlockSpec((H, G3), lambda t: (0, 0)),
            pl.BlockSpec((H, G3), lambda t: (0, 0)),
            pl.BlockSpec((2, 1, H), lambda t: (0, 0, 0)),
        ],
        out_specs=[
            pl.BlockSpec((B, TB, H), lambda t: (0, t, 0)),
            pl.BlockSpec((B, TB, H), lambda t: (0, NTB - 1 - t, 0)),
        ],
        out_shape=[
            jax.ShapeDtypeStruct((B, Lp, H), _f32),
            jax.ShapeDtypeStruct((B, Lp, H), _f32),
        ],
        scratch_shapes=[pltpu.VMEM((2 * B, H), _f32)],
    )(hs0, Gi, Gi, Whf, Whb, bn2)

    out = jnp.concatenate([ysf, ysb], axis=-1)
    hn = jnp.stack([ysf[:, -1, :], ysb[:, 0, :]], axis=0)
    return out, hn


# speedup vs baseline: 1.3775x; 1.1010x over previous
import jax
import jax.numpy as jnp
from jax.experimental import pallas as pl
from jax.experimental.pallas import tpu as pltpu

B = 4
L = 2048
E = 64
H = 512
VOCAB = 512
S = 4
HW = 2100
HWP = 2176
RHP = 512
N_RES = 4
EPS = 1e-05
Lp = L // S
KW = 8
T = 512
NT = L // T
TB = 32
NTB = Lp // TB
G3 = 3 * H

_f32 = jnp.float32


PR = T // S + 2


def _conv_kernel(xw_ref, emb_ref, w_ref, b_ref, out_ref):
    idx = xw_ref[0]
    oh = (idx == jax.lax.broadcasted_iota(jnp.int32, (S * PR, VOCAB), 1))
    xe = jnp.dot(oh.astype(_f32), emb_ref[:],
                 preferred_element_type=_f32)
    m = None
    for p in range(S):
        parts = []
        for d in range(KW):
            q = (p - d) % S
            s = (p - d - q) // S
            parts.append(xe[q * PR + 2 + s: q * PR + 2 + s + T // S])
        xwin = jnp.concatenate(parts, axis=1)
        y = jnp.maximum(
            jnp.dot(xwin.astype(jnp.bfloat16), w_ref[:],
                    preferred_element_type=_f32) + b_ref[:],
            0.0)
        m = y if m is None else jnp.maximum(m, y)
    out_ref[0] = m


def _res_kernel(y_ref, w1_ref, b1_ref, g_ref, bt_ref, w2_ref, b2_ref, out_ref):
    y = y_ref[:]
    for i in range(N_RES):
        r = jnp.maximum(y, 0.0).astype(jnp.bfloat16)
        r = jnp.dot(r, w1_ref[i], preferred_element_type=_f32) + b1_ref[i]
        r = jnp.maximum(r, 0.0)
        r = (r * g_ref[i] + bt_ref[i]).astype(jnp.bfloat16)
        y = y + jnp.dot(r, w2_ref[i], preferred_element_type=_f32) + b2_ref[i]
    out_ref[:] = y


def _proj_kernel(y_ref, w_ref, b_ref, out_ref):
    out_ref[:] = (jnp.dot(y_ref[:].astype(jnp.bfloat16), w_ref[:],
                          preferred_element_type=_f32)
                  + b_ref[:]).astype(jnp.bfloat16)


def _gru_kernel(hs0_ref, gif_ref, gib_ref, whf_ref, whb_ref, bn_ref,
                outf_ref, outb_ref, hs):
    @pl.when(pl.program_id(0) == 0)
    def _():
        hs[:] = hs0_ref[:]

    h = hs[:]
    hf = h[0:B]
    hb = h[B:2 * B]
    whf = whf_ref[:]
    whb = whb_ref[:]
    bnf = bn_ref[0]
    bnb = bn_ref[1]

    def gates(gi_raw, mm, bn, hprev):
        gi = gi_raw.astype(_f32)
        rg = jax.nn.sigmoid(gi[:, 0:H] + mm[:, 0:H])
        zg = jax.nn.sigmoid(gi[:, H:2 * H] + mm[:, H:2 * H])
        ng = jnp.tanh(gi[:, 2 * H:3 * H] + rg * (mm[:, 2 * H:3 * H] + bn))
        return (1.0 - zg) * ng + zg * hprev

    for i in range(TB):
        mmf = jnp.dot(hf.astype(jnp.bfloat16), whf, preferred_element_type=_f32)
        mmb = jnp.dot(hb.astype(jnp.bfloat16), whb, preferred_element_type=_f32)
        hf = gates(gif_ref[:, i, :], mmf, bnf, hf)
        hb = gates(gib_ref[:, TB - 1 - i, :], mmb, bnb, hb)
        outf_ref[:, i, :] = hf
        outb_ref[:, TB - 1 - i, :] = hb
    hs[:] = jnp.concatenate([hf, hb], axis=0)


def kernel(x, h, emb, conv_params, res_params, gru_params):
    Wc = jnp.zeros((KW * E, HWP), _f32)
    bc = jnp.zeros((1, HWP), _f32)
    off = 0
    for i, (W, b) in enumerate(conv_params):
        nf = W.shape[0]
        for d in range(i + 1):
            Wc = Wc.at[d * E:(d + 1) * E, off:off + nf].set(W[:, 0, i - d, :].T)
        bc = bc.at[0, off:off + nf].set(b)
        off += nf
    Wc = Wc.astype(jnp.bfloat16)

    xp = jnp.pad(x.astype(jnp.int32), ((0, 0), (KW, 0)), constant_values=VOCAB)
    arr = xp.reshape(B, (L + KW) // S, S)
    xw = jnp.stack([arr[:, (T // S) * n:(T // S) * n + PR] for n in range(NT)],
                   axis=1)
    xw = xw.transpose(0, 1, 3, 2).reshape(B * NT, S * PR, 1)

    Yp = pl.pallas_call(
        _conv_kernel,
        grid=(B * NT,),
        in_specs=[
            pl.BlockSpec((1, S * PR, 1), lambda g: (g, 0, 0)),
            pl.BlockSpec((VOCAB, E), lambda g: (0, 0)),
            pl.BlockSpec((KW * E, HWP), lambda g: (0, 0)),
            pl.BlockSpec((1, HWP), lambda g: (0, 0)),
        ],
        out_specs=pl.BlockSpec((1, T // S, HWP), lambda g: (g, 0, 0)),
        out_shape=jax.ShapeDtypeStruct((B * NT, T // S, HWP), _f32),
    )(xw, emb, Wc, bc)
    Yf = Yp.reshape(B * Lp, HWP)

    gm = 1.0 / jnp.sqrt(1.0 + EPS)
    w1 = jnp.stack([jnp.zeros((HWP, RHP), _f32).at[:HW, :400].set(p[0].T)
                    for p in res_params]).astype(jnp.bfloat16)
    b1 = jnp.stack([jnp.zeros((1, RHP), _f32).at[0, :400].set(p[1])
                    for p in res_params])
    gmul = jnp.stack([jnp.zeros((1, RHP), _f32).at[0, :400].set(p[4] * gm)
                      for p in res_params])
    beta = jnp.stack([jnp.zeros((1, RHP), _f32).at[0, :400].set(p[5])
                      for p in res_params])
    w2 = jnp.stack([jnp.zeros((RHP, HWP), _f32).at[:400, :HW].set(p[2].T)
                    for p in res_params]).astype(jnp.bfloat16)
    b2 = jnp.stack([jnp.zeros((1, HWP), _f32).at[0, :HW].set(p[3])
                    for p in res_params])

    RM = 256
    Yr = pl.pallas_call(
        _res_kernel,
        grid=(B * Lp // RM,),
        in_specs=[
            pl.BlockSpec((RM, HWP), lambda m: (m, 0)),
            pl.BlockSpec((N_RES, HWP, RHP), lambda m: (0, 0, 0)),
            pl.BlockSpec((N_RES, 1, RHP), lambda m: (0, 0, 0)),
            pl.BlockSpec((N_RES, 1, RHP), lambda m: (0, 0, 0)),
            pl.BlockSpec((N_RES, 1, RHP), lambda m: (0, 0, 0)),
            pl.BlockSpec((N_RES, RHP, HWP), lambda m: (0, 0, 0)),
            pl.BlockSpec((N_RES, 1, HWP), lambda m: (0, 0, 0)),
        ],
        out_specs=pl.BlockSpec((RM, HWP), lambda m: (m, 0)),
        out_shape=jax.ShapeDtypeStruct((B * Lp, HWP), _f32),
    )(Yf, w1, b1, gmul, beta, w2, b2)

    Wih_f, Whh_f, bih_f, bhh_f = gru_params[0]
    Wih_b, Whh_b, bih_b, bhh_b = gru_params[1]
    Wih = jnp.concatenate(
        [jnp.zeros((HWP, G3), _f32).at[:HW, :].set(Wih_f.T),
         jnp.zeros((HWP, G3), _f32).at[:HW, :].set(Wih_b.T)],
        axis=1).astype(jnp.bfloat16)
    zH = jnp.zeros((H,), _f32)
    bih = jnp.concatenate(
        [bih_f + jnp.concatenate([bhh_f[0:2 * H], zH]),
         bih_b + jnp.concatenate([bhh_b[0:2 * H], zH])])[None, :]

    Gi = pl.pallas_call(
        _proj_kernel,
        grid=(B * Lp // RM,),
        in_specs=[
            pl.BlockSpec((RM, HWP), lambda m: (m, 0)),
            pl.BlockSpec((HWP, 2 * G3), lambda m: (0, 0)),
            pl.BlockSpec((1, 2 * G3), lambda m: (0, 0)),
        ],
        out_specs=pl.BlockSpec((RM, 2 * G3), lambda m: (m, 0)),
        out_shape=jax.ShapeDtypeStruct((B * Lp, 2 * G3), jnp.bfloat16),
    )(Yr, Wih, bih)
    Gi = Gi.reshape(B, Lp, 2 * G3)

    Whf = Whh_f.T.astype(jnp.bfloat16)
    Whb = Whh_b.T.astype(jnp.bfloat16)
    bn2 = jnp.stack([bhh_f[2 * H:3 * H][None, :],
                     bhh_b[2 * H:3 * H][None, :]])
    hs0 = jnp.concatenate([h[0], h[1]], axis=0)

    ysf, ysb = pl.pallas_call(
        _gru_kernel,
        grid=(NTB,),
        in_specs=[
            pl.BlockSpec((2 * B, H), lambda t: (0, 0)),
            pl.BlockSpec((B, TB, G3), lambda t: (0, t, 0)),
            pl.BlockSpec((B, TB, G3), lambda t: (0, NTB - 1 - t, 1)),
            pl.BlockSpec((H, G3), lambda t: (0, 0)),
            pl.BlockSpec((H, G3), lambda t: (0, 0)),
            pl.BlockSpec((2, 1, H), lambda t: (0, 0, 0)),
        ],
        out_specs=[
            pl.BlockSpec((B, TB, H), lambda t: (0, t, 0)),
            pl.BlockSpec((B, TB, H), lambda t: (0, NTB - 1 - t, 0)),
        ],
        out_shape=[
            jax.ShapeDtypeStruct((B, Lp, H), _f32),
            jax.ShapeDtypeStruct((B, Lp, H), _f32),
        ],
        scratch_shapes=[pltpu.VMEM((2 * B, H), _f32)],
    )(hs0, Gi, Gi, Whf, Whb, bn2)

    out = jnp.concatenate([ysf, ysb], axis=-1)
    hn = jnp.stack([ysf[:, -1, :], ysb[:, 0, :]], axis=0)
    return out, hn


# speedup vs baseline: 1.3977x; 1.0147x over previous
import jax
import jax.numpy as jnp
from jax.experimental import pallas as pl
from jax.experimental.pallas import tpu as pltpu

B = 4
L = 2048
E = 64
H = 512
VOCAB = 512
S = 4
HW = 2100
HWP = 2176
RHP = 512
N_RES = 4
EPS = 1e-05
Lp = L // S
KW = 8
T = 512
NT = L // T
TB = 32
NTB = Lp // TB
G3 = 3 * H

_f32 = jnp.float32


PR = T // S + 2


def _conv_kernel(xw_ref, emb_ref, w_ref, b_ref, out_ref):
    idx = xw_ref[0]
    oh = (idx == jax.lax.broadcasted_iota(jnp.int32, (S * PR, VOCAB), 1))
    xe = jnp.dot(oh.astype(_f32), emb_ref[:],
                 preferred_element_type=_f32)
    m = None
    for p in range(S):
        parts = []
        for d in range(KW):
            q = (p - d) % S
            s = (p - d - q) // S
            parts.append(xe[q * PR + 2 + s: q * PR + 2 + s + T // S])
        xwin = jnp.concatenate(parts, axis=1)
        y = jnp.maximum(
            jnp.dot(xwin.astype(jnp.bfloat16), w_ref[:],
                    preferred_element_type=_f32) + b_ref[:],
            0.0)
        m = y if m is None else jnp.maximum(m, y)
    out_ref[0] = m.astype(jnp.bfloat16)


def _res_kernel(y_ref, w1_ref, b1_ref, g_ref, bt_ref, w2_ref, b2_ref,
                wih_ref, bih_ref, out_ref):
    y = y_ref[:].astype(_f32)
    for i in range(N_RES):
        r = jnp.maximum(y, 0.0).astype(jnp.bfloat16)
        r = jnp.dot(r, w1_ref[i], preferred_element_type=_f32) + b1_ref[i]
        r = jnp.maximum(r, 0.0)
        r = (r * g_ref[i] + bt_ref[i]).astype(jnp.bfloat16)
        y = y + jnp.dot(r, w2_ref[i], preferred_element_type=_f32) + b2_ref[i]
    out_ref[:] = (jnp.dot(y.astype(jnp.bfloat16), wih_ref[:],
                          preferred_element_type=_f32)
                  + bih_ref[:]).astype(jnp.bfloat16)


def _gru_kernel(hs0_ref, gif_ref, gib_ref, whf_ref, whb_ref, bn_ref,
                outf_ref, outb_ref, hs):
    @pl.when(pl.program_id(0) == 0)
    def _():
        hs[:] = hs0_ref[:]

    h = hs[:]
    hf = h[0:B]
    hb = h[B:2 * B]
    whf = whf_ref[:]
    whb = whb_ref[:]
    bnf = bn_ref[0]
    bnb = bn_ref[1]

    def gates(gi_raw, mm, bn, hprev):
        gi = gi_raw.astype(_f32)
        rg = jax.nn.sigmoid(gi[:, 0:H] + mm[:, 0:H])
        zg = jax.nn.sigmoid(gi[:, H:2 * H] + mm[:, H:2 * H])
        ng = jnp.tanh(gi[:, 2 * H:3 * H] + rg * (mm[:, 2 * H:3 * H] + bn))
        return (1.0 - zg) * ng + zg * hprev

    for i in range(TB):
        mmf = jnp.dot(hf.astype(jnp.bfloat16), whf, preferred_element_type=_f32)
        mmb = jnp.dot(hb.astype(jnp.bfloat16), whb, preferred_element_type=_f32)
        hf = gates(gif_ref[:, i, :], mmf, bnf, hf)
        hb = gates(gib_ref[:, TB - 1 - i, :], mmb, bnb, hb)
        outf_ref[:, i, :] = hf
        outb_ref[:, TB - 1 - i, :] = hb
    hs[:] = jnp.concatenate([hf, hb], axis=0)


def kernel(x, h, emb, conv_params, res_params, gru_params):
    Wc = jnp.zeros((KW * E, HWP), _f32)
    bc = jnp.zeros((1, HWP), _f32)
    off = 0
    for i, (W, b) in enumerate(conv_params):
        nf = W.shape[0]
        for d in range(i + 1):
            Wc = Wc.at[d * E:(d + 1) * E, off:off + nf].set(W[:, 0, i - d, :].T)
        bc = bc.at[0, off:off + nf].set(b)
        off += nf
    Wc = Wc.astype(jnp.bfloat16)

    xp = jnp.pad(x.astype(jnp.int32), ((0, 0), (KW, 0)), constant_values=VOCAB)
    arr = xp.reshape(B, (L + KW) // S, S)
    xw = jnp.stack([arr[:, (T // S) * n:(T // S) * n + PR] for n in range(NT)],
                   axis=1)
    xw = xw.transpose(0, 1, 3, 2).reshape(B * NT, S * PR, 1)

    Yp = pl.pallas_call(
        _conv_kernel,
        grid=(B * NT,),
        in_specs=[
            pl.BlockSpec((1, S * PR, 1), lambda g: (g, 0, 0)),
            pl.BlockSpec((VOCAB, E), lambda g: (0, 0)),
            pl.BlockSpec((KW * E, HWP), lambda g: (0, 0)),
            pl.BlockSpec((1, HWP), lambda g: (0, 0)),
        ],
        out_specs=pl.BlockSpec((1, T // S, HWP), lambda g: (g, 0, 0)),
        out_shape=jax.ShapeDtypeStruct((B * NT, T // S, HWP), jnp.bfloat16),
    )(xw, emb, Wc, bc)
    Yf = Yp.reshape(B * Lp, HWP)

    gm = 1.0 / jnp.sqrt(1.0 + EPS)
    w1 = jnp.stack([jnp.zeros((HWP, RHP), _f32).at[:HW, :400].set(p[0].T)
                    for p in res_params]).astype(jnp.bfloat16)
    b1 = jnp.stack([jnp.zeros((1, RHP), _f32).at[0, :400].set(p[1])
                    for p in res_params])
    gmul = jnp.stack([jnp.zeros((1, RHP), _f32).at[0, :400].set(p[4] * gm)
                      for p in res_params])
    beta = jnp.stack([jnp.zeros((1, RHP), _f32).at[0, :400].set(p[5])
                      for p in res_params])
    w2 = jnp.stack([jnp.zeros((RHP, HWP), _f32).at[:400, :HW].set(p[2].T)
                    for p in res_params]).astype(jnp.bfloat16)
    b2 = jnp.stack([jnp.zeros((1, HWP), _f32).at[0, :HW].set(p[3])
                    for p in res_params])

    Wih_f, Whh_f, bih_f, bhh_f = gru_params[0]
    Wih_b, Whh_b, bih_b, bhh_b = gru_params[1]
    Wih = jnp.concatenate(
        [jnp.zeros((HWP, G3), _f32).at[:HW, :].set(Wih_f.T),
         jnp.zeros((HWP, G3), _f32).at[:HW, :].set(Wih_b.T)],
        axis=1).astype(jnp.bfloat16)
    zH = jnp.zeros((H,), _f32)
    bih = jnp.concatenate(
        [bih_f + jnp.concatenate([bhh_f[0:2 * H], zH]),
         bih_b + jnp.concatenate([bhh_b[0:2 * H], zH])])[None, :]

    RM = 256
    Gi = pl.pallas_call(
        _res_kernel,
        grid=(B * Lp // RM,),
        in_specs=[
            pl.BlockSpec((RM, HWP), lambda m: (m, 0)),
            pl.BlockSpec((N_RES, HWP, RHP), lambda m: (0, 0, 0)),
            pl.BlockSpec((N_RES, 1, RHP), lambda m: (0, 0, 0)),
            pl.BlockSpec((N_RES, 1, RHP), lambda m: (0, 0, 0)),
            pl.BlockSpec((N_RES, 1, RHP), lambda m: (0, 0, 0)),
            pl.BlockSpec((N_RES, RHP, HWP), lambda m: (0, 0, 0)),
            pl.BlockSpec((N_RES, 1, HWP), lambda m: (0, 0, 0)),
            pl.BlockSpec((HWP, 2 * G3), lambda m: (0, 0)),
            pl.BlockSpec((1, 2 * G3), lambda m: (0, 0)),
        ],
        out_specs=pl.BlockSpec((RM, 2 * G3), lambda m: (m, 0)),
        out_shape=jax.ShapeDtypeStruct((B * Lp, 2 * G3), jnp.bfloat16),
    )(Yf, w1, b1, gmul, beta, w2, b2, Wih, bih)
    Gi = Gi.reshape(B, Lp, 2 * G3)

    Whf = Whh_f.T.astype(jnp.bfloat16)
    Whb = Whh_b.T.astype(jnp.bfloat16)
    bn2 = jnp.stack([bhh_f[2 * H:3 * H][None, :],
                     bhh_b[2 * H:3 * H][None, :]])
    hs0 = jnp.concatenate([h[0], h[1]], axis=0)

    ysf, ysb = pl.pallas_call(
        _gru_kernel,
        grid=(NTB,),
        in_specs=[
            pl.BlockSpec((2 * B, H), lambda t: (0, 0)),
            pl.BlockSpec((B, TB, G3), lambda t: (0, t, 0)),
            pl.BlockSpec((B, TB, G3), lambda t: (0, NTB - 1 - t, 1)),
            pl.BlockSpec((H, G3), lambda t: (0, 0)),
            pl.BlockSpec((H, G3), lambda t: (0, 0)),
            pl.BlockSpec((2, 1, H), lambda t: (0, 0, 0)),
        ],
        out_specs=[
            pl.BlockSpec((B, TB, H), lambda t: (0, t, 0)),
            pl.BlockSpec((B, TB, H), lambda t: (0, NTB - 1 - t, 0)),
        ],
        out_shape=[
            jax.ShapeDtypeStruct((B, Lp, H), _f32),
            jax.ShapeDtypeStruct((B, Lp, H), _f32),
        ],
        scratch_shapes=[pltpu.VMEM((2 * B, H), _f32)],
    )(hs0, Gi, Gi, Whf, Whb, bn2)

    out = jnp.concatenate([ysf, ysb], axis=-1)
    hn = jnp.stack([ysf[:, -1, :], ysb[:, 0, :]], axis=0)
    return out, hn
